# Initial kernel scaffold; baseline (speedup 1.0000x reference)
import jax
import jax.numpy as jnp
from jax.experimental import pallas as pl


def kernel(xyzt, point_features, box_features, frame2batchidx, point2frameidx, params):
    raise NotImplementedError("write your pallas kernel here")


# speedup vs baseline: 12.7820x; 12.7820x over previous
import functools

import jax
import jax.numpy as jnp
import numpy as np
from jax.experimental import pallas as pl
from jax.experimental.pallas import tpu as pltpu

_B, _T, _N_PER_FRAME = 4, 4, 2048
_BT = _B * _T
_BTN = _BT * _N_PER_FRAME
_SUB = 8
_Q_PER_FRAME = _N_PER_FRAME // _SUB
_NQ = _BT * _Q_PER_FRAME
_Q_PER_BATCH = _T * _Q_PER_FRAME
_FEAT = 64
_D_PRE = 128
_D = 256
_D_FF = 1024
_OUT = 256
_N_HEADS = 4
_D_H = _D // _N_HEADS
_TIME_WINDOW = 1.5


def _f32dot(a, b):
    return jax.lax.dot_general(a, b, (((1,), (0,)), ((), ())),
                               preferred_element_type=jnp.float32)


def _layernorm(x, g, b):
    m = jnp.mean(x, axis=-1, keepdims=True)
    xc = x - m
    v = jnp.mean(xc * xc, axis=-1, keepdims=True)
    return xc * jax.lax.rsqrt(v + 1e-5) * g + b


def _encoder_body(qin_ref, pe_ref,
                  w_pre1, b_pre1, w_pre2, b_pre2, b_fourier,
                  w_cat, b_cat, w_pos, b_pos,
                  wq, wk, wv, wo, ln1_g, ln1_b,
                  w_ff1, b_ff1, w_ff2, b_ff2, ln2_g, ln2_b,
                  w_proj1, b_proj1, w_proj2, b_proj2,
                  enc_ref):
    h = jax.nn.relu(_f32dot(qin_ref[:], w_pre1[:]) + b_pre1[:])
    qf = jax.nn.relu(_f32dot(h, w_pre2[:]) + b_pre2[:])
    proj = _f32dot(pe_ref[:], b_fourier[:])
    four = jnp.concatenate([jnp.sin(proj), jnp.cos(proj)], axis=1)
    pos = _f32dot(four, w_pos[:]) + b_pos[:]
    cat = _f32dot(four, w_cat[:]) + b_cat[:]
    feats = jnp.concatenate([qf, cat], axis=1) + pos

    q = _f32dot(feats, wq[:])
    k = _f32dot(feats, wk[:])
    v = _f32dot(feats, wv[:])
    heads = []
    scale = 1.0 / np.sqrt(_D_H).astype(np.float32)
    for hd in range(_N_HEADS):
        sl = slice(hd * _D_H, (hd + 1) * _D_H)
        qh, kh, vh = q[:, sl], k[:, sl], v[:, sl]
        s = jax.lax.dot_general(qh, kh, (((1,), (1,)), ((), ())),
                                preferred_element_type=jnp.float32) * scale
        s = s - jnp.max(s, axis=1, keepdims=True)
        e = jnp.exp(s)
        a = e / jnp.sum(e, axis=1, keepdims=True)
        heads.append(_f32dot(a, vh))
    o = jnp.concatenate(heads, axis=1)

    h1 = _layernorm(feats + _f32dot(o, wo[:]), ln1_g[:], ln1_b[:])
    ff = _f32dot(jax.nn.relu(_f32dot(h1, w_ff1[:]) + b_ff1[:]), w_ff2[:]) + b_ff2[:]
    h2 = _layernorm(h1 + ff, ln2_g[:], ln2_b[:])
    e1 = jax.nn.relu(_f32dot(h2, w_proj1[:]) + b_proj1[:])
    enc_ref[:] = jax.nn.relu(_f32dot(e1, w_proj2[:]) + b_proj2[:])


def _interp_body(pxyz_ref, qxyz_ref, qfeat_ref, w_fp1, b_fp1, w_fp2, b_fp2,
                 out_ref):
    p = pxyz_ref[:]
    qx = qxyz_ref[:]
    pn = jnp.sum(p * p, axis=1, keepdims=True)
    qn = jnp.sum(qx * qx, axis=1, keepdims=True)
    cross = jax.lax.dot_general(p, qx, (((1,), (1,)), ((), ())),
                                preferred_element_type=jnp.float32)
    d2 = pn + qn.T - 2.0 * cross
    neg = -d2
    cols = jax.lax.broadcasted_iota(jnp.int32, d2.shape, 1)
    wmat = jnp.zeros(d2.shape, jnp.float32)
    wsum = jnp.zeros((d2.shape[0], 1), jnp.float32)
    for _ in range(3):
        m = jnp.max(neg, axis=1, keepdims=True)
        idx = jnp.min(jnp.where(neg == m, cols, _Q_PER_FRAME), axis=1,
                      keepdims=True)
        sel = cols == idx
        dist = jnp.sqrt(jnp.maximum(-m, 1e-10))
        wk = 1.0 / (dist + 1e-8)
        wmat = wmat + jnp.where(sel, wk, 0.0)
        wsum = wsum + wk
        neg = jnp.where(sel, -jnp.inf, neg)
    wmat = wmat / wsum
    interp = _f32dot(wmat, qfeat_ref[:])
    g = jax.nn.relu(_f32dot(interp, w_fp1[:]) + b_fp1[:])
    out_ref[:] = jax.nn.relu(_f32dot(g, w_fp2[:]) + b_fp2[:])


def _full(shape):
    nd = len(shape)
    return pl.BlockSpec(shape, lambda i, *, _nd=nd: (0,) * _nd)


def kernel(xyzt, point_features, box_features, frame2batchidx, point2frameidx,
           params):
    pr = params
    xyz = xyzt[:, :3]
    pts = jnp.concatenate([xyzt, point_features, box_features], axis=1)
    q_all = pts.reshape(_NQ, _SUB, 73)[:, 0, :]
    qin = jnp.pad(q_all, ((0, 0), (0, 128 - 73)))
    w_pre1 = jnp.pad(pr['W_pre1'], ((0, 128 - 73), (0, 0)))
    pe_in = jnp.concatenate(
        [q_all[:, 0:3], q_all[:, 3:4] / _TIME_WINDOW, q_all[:, 68:73]], axis=1)
    pe = jnp.pad(pe_in, ((0, 0), (0, 128 - 9)))
    b_fourier = jnp.pad(pr['B_fourier'], ((0, 128 - 9), (0, 0)))

    def row(x):
        return x.reshape(1, -1)

    enc_weights = [
        w_pre1, row(pr['b_pre1']), pr['W_pre2'], row(pr['b_pre2']), b_fourier,
        pr['W_cat'], row(pr['b_cat']), pr['W_pos'], row(pr['b_pos']),
        pr['Wq'], pr['Wk'], pr['Wv'], pr['Wo'],
        row(pr['ln1_g']), row(pr['ln1_b']),
        pr['W_ff1'], row(pr['b_ff1']), pr['W_ff2'], row(pr['b_ff2']),
        row(pr['ln2_g']), row(pr['ln2_b']),
        pr['W_proj1'], row(pr['b_proj1']), pr['W_proj2'], row(pr['b_proj2']),
    ]

    enc_features = pl.pallas_call(
        _encoder_body,
        grid=(_B,),
        in_specs=[
            pl.BlockSpec((_Q_PER_BATCH, 128), lambda b: (b, 0)),
            pl.BlockSpec((_Q_PER_BATCH, 128), lambda b: (b, 0)),
        ] + [_full(w.shape) for w in enc_weights],
        out_specs=pl.BlockSpec((_Q_PER_BATCH, _D), lambda b: (b, 0)),
        out_shape=jax.ShapeDtypeStruct((_NQ, _D), jnp.float32),
    )(qin, pe, *enc_weights)

    xyz8 = jnp.pad(xyz, ((0, 0), (0, 5)))
    qxyz8 = xyz8.reshape(_NQ, _SUB, 8)[:, 0, :]

    interp_weights = [row(pr['b_fp1']), row(pr['b_fp2'])]
    per_point_feats = pl.pallas_call(
        _interp_body,
        grid=(_BT,),
        in_specs=[
            pl.BlockSpec((_N_PER_FRAME, 8), lambda f: (f, 0)),
            pl.BlockSpec((_Q_PER_FRAME, 8), lambda f: (f, 0)),
            pl.BlockSpec((_Q_PER_FRAME, _OUT), lambda f: (f, 0)),
            _full(pr['W_fp1'].shape),
            _full((1, _OUT)),
            _full(pr['W_fp2'].shape),
            _full((1, _OUT)),
        ],
        out_specs=pl.BlockSpec((_N_PER_FRAME, _OUT), lambda f: (f, 0)),
        out_shape=jax.ShapeDtypeStruct((_BTN, _OUT), jnp.float32),
    )(xyz8, qxyz8, enc_features, pr['W_fp1'], interp_weights[0],
      pr['W_fp2'], interp_weights[1])

    return per_point_feats, enc_features
